# baseline (speedup 1.0000x reference)
import jax
import jax.numpy as jnp
from jax.experimental import pallas as pl


def _passthrough(img_ref, lab_ref, img_out, lab_out):
    img_out[...] = img_ref[...]
    lab_out[...] = lab_ref[...]


def kernel(images, labels):
    B, H, W, C = images.shape
    img2 = images.reshape(B * H, W * C)
    rows, cols = img2.shape
    grid = 8
    blk = rows // grid
    out_img, out_lab = pl.pallas_call(
        _passthrough,
        grid=(grid,),
        in_specs=[
            pl.BlockSpec((blk, cols), lambda i: (i, 0)),
            pl.BlockSpec(labels.shape, lambda i: (0, 0)),
        ],
        out_specs=[
            pl.BlockSpec((blk, cols), lambda i: (i, 0)),
            pl.BlockSpec(labels.shape, lambda i: (0, 0)),
        ],
        out_shape=[
            jax.ShapeDtypeStruct(img2.shape, img2.dtype),
            jax.ShapeDtypeStruct(labels.shape, labels.dtype),
        ],
    )(img2, labels)
    return (out_img.reshape(B, H, W, C), out_lab)
